# baseline (speedup 1.0000x reference)
import jax
import jax.numpy as jnp
import numpy as np
from jax.experimental import pallas as pl
from jax.experimental.pallas import tpu as pltpu
from jax.experimental.pallas import tpu_sc as plsc

NF = 26
D = 64
NPAIR = NF * (NF - 1) // 2
BLK = 512
CHUNK = 16


def _tc_body(x_ref, o_ref):
    x3 = x_ref[...].reshape(NF, D, BLK)
    off = 0
    for i in range(NF - 1):
        nj = NF - 1 - i
        q = x3[i + 1:]
        p = x3[i]
        acc = q[:, 0:8, :] * p[None, 0:8, :]
        for dv in range(1, D // 8):
            sl = slice(dv * 8, dv * 8 + 8)
            acc = acc + q[:, sl, :] * p[None, sl, :]
        o_ref[off:off + nj, :] = jnp.sum(acc, axis=1)
        off += nj


def _tc_part(x2d):
    b = x2d.shape[0]
    xt = x2d.T.astype(jnp.bfloat16)
    out_t = pl.pallas_call(
        _tc_body,
        grid=(b // BLK,),
        in_specs=[pl.BlockSpec((NF * D, BLK), lambda i: (0, i))],
        out_specs=pl.BlockSpec((NPAIR, BLK), lambda i: (0, i)),
        out_shape=jax.ShapeDtypeStruct((NPAIR, b), jnp.bfloat16),
    )(xt)
    return out_t.T.astype(jnp.float32)


SC_LANES = 128
N_WORKERS = 32
D2 = D // 2
OROWS = (NPAIR + 1) // 2


def _sc_part(x2d):
    b = x2d.shape[0]
    nchunk = b // SC_LANES
    per_worker = nchunk // N_WORKERS
    x4 = x2d.reshape(nchunk, SC_LANES, NF, D2, 2).transpose(0, 2, 3, 4, 1)
    x4 = x4.reshape(nchunk, NF * D2, 2, SC_LANES).astype(jnp.bfloat16)

    @pl.kernel(
        out_type=jax.ShapeDtypeStruct((nchunk, 2 * OROWS, 2, SC_LANES),
                                      jnp.bfloat16),
        mesh=plsc.VectorSubcoreMesh(core_axis_name="c", subcore_axis_name="s"),
        scratch_types=[
            pltpu.VMEM((NF * D2, 2, SC_LANES), jnp.bfloat16),
            pltpu.VMEM((OROWS, 2, SC_LANES), jnp.bfloat16),
            pltpu.SemaphoreType.DMA,
        ],
    )
    def run(x_hbm, o_hbm, x_vmem, o_vmem, sem):
        c = jax.lax.axis_index("c")
        s = jax.lax.axis_index("s")
        wid = c * 16 + s

        @pl.loop(0, per_worker)
        def _(t):
            chunk = wid * per_worker + t
            pltpu.async_copy(x_hbm.at[chunk], x_vmem, sem).wait()

            for p in range(2):
                @pl.loop(0, NF - 1)
                def _(i, _p=p):
                    @pl.loop(1, NF)
                    def _(j, _p=_p, _i=i):
                        row = ((_i * (2 * NF - 1 - _i)) // 2 + (j - _i - 1)
                               - _p * OROWS)

                        @pl.when((j > _i) & (row >= 0) & (row < OROWS))
                        def _():
                            xi = x_vmem.at[pl.ds(_i * D2, D2)]
                            xj = x_vmem.at[pl.ds(j * D2, D2)]
                            for g in range(SC_LANES // 16):
                                sl = pl.ds(g * 16, 16)
                                terms = [xi[d, :, sl] * xj[d, :, sl]
                                         for d in range(D2)]
                                while len(terms) > 1:
                                    terms = [a + b for a, b in
                                             zip(terms[::2], terms[1::2])]
                                o_vmem[row, :, sl] = terms[0]

                pltpu.async_copy(
                    o_vmem, o_hbm.at[chunk, pl.ds(p * OROWS, OROWS)], sem
                ).wait()

    out = run(x4)
    out = jnp.sum(out.astype(jnp.float32), axis=2)
    return out[:, :NPAIR, :].transpose(0, 2, 1).reshape(b, NPAIR)


def kernel(x):
    b = x.shape[0]
    x2d = x.reshape(b, NF * D)
    return _sc_part(x2d)


# speedup vs baseline: 1.0689x; 1.0689x over previous
import jax
import jax.numpy as jnp
import numpy as np
from jax.experimental import pallas as pl
from jax.experimental.pallas import tpu as pltpu
from jax.experimental.pallas import tpu_sc as plsc

NF = 26
D = 64
NPAIR = NF * (NF - 1) // 2
BLK = 512
CHUNK = 16


def _tc_body(x_ref, o_ref):
    x3 = x_ref[...].reshape(NF, D, BLK)
    off = 0
    for i in range(NF - 1):
        nj = NF - 1 - i
        q = x3[i + 1:]
        p = x3[i]
        acc = q[:, 0:8, :] * p[None, 0:8, :]
        for dv in range(1, D // 8):
            sl = slice(dv * 8, dv * 8 + 8)
            acc = acc + q[:, sl, :] * p[None, sl, :]
        o_ref[off:off + nj, :] = jnp.sum(acc, axis=1)
        off += nj


def _tc_part(x2d):
    b = x2d.shape[0]
    xt = x2d.T.astype(jnp.bfloat16)
    out_t = pl.pallas_call(
        _tc_body,
        grid=(b // BLK,),
        in_specs=[pl.BlockSpec((NF * D, BLK), lambda i: (0, i))],
        out_specs=pl.BlockSpec((NPAIR, BLK), lambda i: (0, i)),
        out_shape=jax.ShapeDtypeStruct((NPAIR, b), jnp.bfloat16),
    )(xt)
    return out_t.T.astype(jnp.float32)


SC_LANES = 128
N_WORKERS = 32
D2 = D // 2
OROWS = (NPAIR + 1) // 2


def _sc_part(x2d):
    b = x2d.shape[0]
    nchunk = b // SC_LANES
    per_worker = nchunk // N_WORKERS
    x4 = x2d.reshape(nchunk, SC_LANES, NF, D2, 2).transpose(0, 2, 3, 4, 1)
    x4 = x4.reshape(nchunk, NF * D2, 2, SC_LANES).astype(jnp.bfloat16)

    @pl.kernel(
        out_type=jax.ShapeDtypeStruct((nchunk, 2 * OROWS, 1, SC_LANES),
                                      jnp.bfloat16),
        mesh=plsc.VectorSubcoreMesh(core_axis_name="c", subcore_axis_name="s"),
        scratch_types=[
            pltpu.VMEM((NF * D2, 2, SC_LANES), jnp.bfloat16),
            pltpu.VMEM((OROWS, 1, SC_LANES), jnp.bfloat16),
            pltpu.SemaphoreType.DMA,
        ],
    )
    def run(x_hbm, o_hbm, x_vmem, o_vmem, sem):
        c = jax.lax.axis_index("c")
        s = jax.lax.axis_index("s")
        wid = c * 16 + s

        @pl.loop(0, per_worker)
        def _(t):
            chunk = wid * per_worker + t
            pltpu.async_copy(x_hbm.at[chunk], x_vmem, sem).wait()

            for p in range(2):
                @pl.loop(0, NF - 1)
                def _(i, _p=p):
                    @pl.loop(1, NF)
                    def _(j, _p=_p, _i=i):
                        row = ((_i * (2 * NF - 1 - _i)) // 2 + (j - _i - 1)
                               - _p * OROWS)

                        @pl.when((j > _i) & (row >= 0) & (row < OROWS))
                        def _():
                            xi = x_vmem.at[pl.ds(_i * D2, D2)]
                            xj = x_vmem.at[pl.ds(j * D2, D2)]
                            for g in range(SC_LANES // 32):
                                sl = pl.ds(g * 32, 32)
                                terms = [xi[d, s_, sl] * xj[d, s_, sl]
                                         for d in range(D2) for s_ in (0, 1)]
                                while len(terms) > 1:
                                    terms = [a + b for a, b in
                                             zip(terms[::2], terms[1::2])]
                                o_vmem[row, 0, sl] = terms[0]

                pltpu.async_copy(
                    o_vmem, o_hbm.at[chunk, pl.ds(p * OROWS, OROWS)], sem
                ).wait()

    out = run(x4)
    out = out.astype(jnp.float32).reshape(nchunk, 2 * OROWS, SC_LANES)
    return out[:, :NPAIR, :].transpose(0, 2, 1).reshape(b, NPAIR)


def kernel(x):
    b = x.shape[0]
    x2d = x.reshape(b, NF * D)
    return _sc_part(x2d)


# speedup vs baseline: 3.3023x; 3.0893x over previous
import jax
import jax.numpy as jnp
import numpy as np
from jax.experimental import pallas as pl
from jax.experimental.pallas import tpu as pltpu
from jax.experimental.pallas import tpu_sc as plsc

NF = 26
D = 64
D2 = D // 2
NPAIR = NF * (NF - 1) // 2
I_CUT = 16
NF_SC = NF - I_CUT
TC_PAIRS = NPAIR - NF_SC * (NF_SC - 1) // 2
SC_PAIRS = NPAIR - TC_PAIRS
BLK = 512
SC_LANES = 128
N_WORKERS = 32


def _tc_body(x_ref, o_ref):
    x3 = x_ref[...].reshape(NF, D, BLK)
    off = 0
    for i in range(I_CUT):
        nj = NF - 1 - i
        q = x3[i + 1:]
        p = x3[i]
        acc = q[:, 0:8, :] * p[None, 0:8, :]
        for dv in range(1, D // 8):
            sl = slice(dv * 8, dv * 8 + 8)
            acc = acc + q[:, sl, :] * p[None, sl, :]
        o_ref[off:off + nj, :] = jnp.sum(acc, axis=1)
        off += nj


def _tc_part(x2d):
    b = x2d.shape[0]
    xt = x2d.T.astype(jnp.bfloat16)
    out_t = pl.pallas_call(
        _tc_body,
        grid=(b // BLK,),
        in_specs=[pl.BlockSpec((NF * D, BLK), lambda i: (0, i))],
        out_specs=pl.BlockSpec((TC_PAIRS, BLK), lambda i: (0, i)),
        out_shape=jax.ShapeDtypeStruct((TC_PAIRS, b), jnp.bfloat16),
    )(xt)
    return out_t.T.astype(jnp.float32)


def _sc_part(x2d):
    b = x2d.shape[0]
    nchunk = b // SC_LANES
    per_worker = nchunk // N_WORKERS
    xs = x2d[:, I_CUT * D:]
    x4 = xs.reshape(nchunk, SC_LANES, NF_SC, D2, 2).transpose(0, 2, 3, 4, 1)
    x4 = x4.reshape(nchunk, NF_SC * D2, 2, SC_LANES).astype(jnp.bfloat16)

    @pl.kernel(
        out_type=jax.ShapeDtypeStruct((nchunk, SC_PAIRS, 1, SC_LANES),
                                      jnp.bfloat16),
        mesh=plsc.VectorSubcoreMesh(core_axis_name="c", subcore_axis_name="s"),
        scratch_types=[
            pltpu.VMEM((NF_SC * D2, 2, SC_LANES), jnp.bfloat16),
            pltpu.VMEM((SC_PAIRS, 1, SC_LANES), jnp.bfloat16),
            pltpu.SemaphoreType.DMA,
        ],
    )
    def run(x_hbm, o_hbm, x_vmem, o_vmem, sem):
        c = jax.lax.axis_index("c")
        s = jax.lax.axis_index("s")
        wid = c * 16 + s

        @pl.loop(0, per_worker)
        def _(t):
            chunk = wid * per_worker + t
            pltpu.async_copy(x_hbm.at[chunk], x_vmem, sem).wait()

            @pl.loop(0, NF_SC - 1)
            def _(i):
                @pl.loop(1, NF_SC)
                def _(j, _i=i):
                    row = (_i * (2 * NF_SC - 1 - _i)) // 2 + (j - _i - 1)

                    @pl.when(j > _i)
                    def _():
                        xi = x_vmem.at[pl.ds(_i * D2, D2)]
                        xj = x_vmem.at[pl.ds(j * D2, D2)]
                        for g in range(SC_LANES // 32):
                            sl = pl.ds(g * 32, 32)
                            terms = [xi[d, s_, sl] * xj[d, s_, sl]
                                     for d in range(D2) for s_ in (0, 1)]
                            while len(terms) > 1:
                                terms = [a + b for a, b in
                                         zip(terms[::2], terms[1::2])]
                            o_vmem[row, 0, sl] = terms[0]

            pltpu.async_copy(o_vmem, o_hbm.at[chunk], sem).wait()

    out = run(x4)
    out = out.astype(jnp.float32).reshape(nchunk, SC_PAIRS, SC_LANES)
    return out.transpose(0, 2, 1).reshape(b, SC_PAIRS)


def kernel(x):
    b = x.shape[0]
    x2d = x.reshape(b, NF * D)
    sc_out = _sc_part(x2d)
    tc_out = _tc_part(x2d)
    return jnp.concatenate([tc_out, sc_out], axis=1)


# speedup vs baseline: 4.2218x; 1.2785x over previous
import jax
import jax.numpy as jnp
import numpy as np
from jax.experimental import pallas as pl
from jax.experimental.pallas import tpu as pltpu
from jax.experimental.pallas import tpu_sc as plsc

NF = 26
D = 64
D2 = D // 2
NPAIR = NF * (NF - 1) // 2
I_CUT = 23
NF_SC = NF - I_CUT
TC_PAIRS = NPAIR - NF_SC * (NF_SC - 1) // 2
SC_PAIRS = NPAIR - TC_PAIRS
BLK = 512
SC_LANES = 128
N_WORKERS = 32


def _tc_body(x_ref, o_ref):
    x3 = x_ref[...].reshape(NF, D, BLK)
    off = 0
    for i in range(I_CUT):
        nj = NF - 1 - i
        q = x3[i + 1:]
        p = x3[i]
        acc = q[:, 0:8, :] * p[None, 0:8, :]
        for dv in range(1, D // 8):
            sl = slice(dv * 8, dv * 8 + 8)
            acc = acc + q[:, sl, :] * p[None, sl, :]
        o_ref[off:off + nj, :] = jnp.sum(acc, axis=1)
        off += nj


def _tc_part(x2d):
    b = x2d.shape[0]
    xt = x2d.T.astype(jnp.bfloat16)
    out_t = pl.pallas_call(
        _tc_body,
        grid=(b // BLK,),
        in_specs=[pl.BlockSpec((NF * D, BLK), lambda i: (0, i))],
        out_specs=pl.BlockSpec((TC_PAIRS, BLK), lambda i: (0, i)),
        out_shape=jax.ShapeDtypeStruct((TC_PAIRS, b), jnp.bfloat16),
    )(xt)
    return out_t.T.astype(jnp.float32)


def _sc_part(x2d):
    b = x2d.shape[0]
    nchunk = b // SC_LANES
    per_worker = nchunk // N_WORKERS
    xs = x2d[:, I_CUT * D:]
    x4 = xs.reshape(nchunk, SC_LANES, NF_SC, D2, 2).transpose(0, 2, 3, 4, 1)
    x4 = x4.reshape(nchunk, NF_SC * D2, 2, SC_LANES).astype(jnp.bfloat16)

    @pl.kernel(
        out_type=jax.ShapeDtypeStruct((nchunk, SC_PAIRS, 1, SC_LANES),
                                      jnp.bfloat16),
        mesh=plsc.VectorSubcoreMesh(core_axis_name="c", subcore_axis_name="s"),
        scratch_types=[
            pltpu.VMEM((NF_SC * D2, 2, SC_LANES), jnp.bfloat16),
            pltpu.VMEM((SC_PAIRS, 1, SC_LANES), jnp.bfloat16),
            pltpu.SemaphoreType.DMA,
        ],
    )
    def run(x_hbm, o_hbm, x_vmem, o_vmem, sem):
        c = jax.lax.axis_index("c")
        s = jax.lax.axis_index("s")
        wid = c * 16 + s

        @pl.loop(0, per_worker)
        def _(t):
            chunk = wid * per_worker + t
            pltpu.async_copy(x_hbm.at[chunk], x_vmem, sem).wait()

            @pl.loop(0, NF_SC - 1)
            def _(i):
                @pl.loop(1, NF_SC)
                def _(j, _i=i):
                    row = (_i * (2 * NF_SC - 1 - _i)) // 2 + (j - _i - 1)

                    @pl.when(j > _i)
                    def _():
                        xi = x_vmem.at[pl.ds(_i * D2, D2)]
                        xj = x_vmem.at[pl.ds(j * D2, D2)]
                        for g in range(SC_LANES // 32):
                            sl = pl.ds(g * 32, 32)
                            terms = [xi[d, s_, sl] * xj[d, s_, sl]
                                     for d in range(D2) for s_ in (0, 1)]
                            while len(terms) > 1:
                                terms = [a + b for a, b in
                                         zip(terms[::2], terms[1::2])]
                            o_vmem[row, 0, sl] = terms[0]

            pltpu.async_copy(o_vmem, o_hbm.at[chunk], sem).wait()

    out = run(x4)
    out = out.astype(jnp.float32).reshape(nchunk, SC_PAIRS, SC_LANES)
    return out.transpose(0, 2, 1).reshape(b, SC_PAIRS)


def kernel(x):
    b = x.shape[0]
    x2d = x.reshape(b, NF * D)
    sc_out = _sc_part(x2d)
    tc_out = _tc_part(x2d)
    return jnp.concatenate([tc_out, sc_out], axis=1)


# speedup vs baseline: 6.2132x; 1.4717x over previous
import jax
import jax.numpy as jnp
import numpy as np
from jax.experimental import pallas as pl

NF = 26
D = 64
NPAIR = NF * (NF - 1) // 2
BLK = 512


def _tc_body(x_ref, o_ref):
    x3 = x_ref[...].reshape(NF, D, BLK)
    off = 0
    for i in range(NF - 1):
        nj = NF - 1 - i
        q = x3[i + 1:]
        p = x3[i]
        acc = q[:, 0:8, :] * p[None, 0:8, :]
        for dv in range(1, D // 8):
            sl = slice(dv * 8, dv * 8 + 8)
            acc = acc + q[:, sl, :] * p[None, sl, :]
        o_ref[off:off + nj, :] = jnp.sum(acc, axis=1)
        off += nj


def kernel(x):
    b = x.shape[0]
    xt = x.reshape(b, NF * D).T.astype(jnp.bfloat16)
    out_t = pl.pallas_call(
        _tc_body,
        grid=(b // BLK,),
        in_specs=[pl.BlockSpec((NF * D, BLK), lambda i: (0, i))],
        out_specs=pl.BlockSpec((NPAIR, BLK), lambda i: (0, i)),
        out_shape=jax.ShapeDtypeStruct((NPAIR, b), jnp.bfloat16),
    )(xt)
    return out_t.T.astype(jnp.float32)
